# baseline (speedup 1.0000x reference)
import jax
import jax.numpy as jnp
from jax.experimental import pallas as pl

N = 10000
F_IN = 128
H = 8
D = 32
HD = H * D
G = 64


def _mm_kernel(x_ref, w_ref, o_ref):
    o_ref[...] = jnp.dot(x_ref[...], w_ref[...], preferred_element_type=jnp.float32)


def _mm(x, w):
    m, k = x.shape
    k2, n = w.shape
    bm = 400
    return pl.pallas_call(
        _mm_kernel,
        grid=(m // bm,),
        in_specs=[pl.BlockSpec((bm, k), lambda i: (i, 0)),
                  pl.BlockSpec((k, n), lambda i: (0, 0))],
        out_specs=pl.BlockSpec((bm, n), lambda i: (i, 0)),
        out_shape=jax.ShapeDtypeStruct((m, n), jnp.float32),
    )(x, w)


def _gmax(x, batch):
    m = jax.ops.segment_max(x, batch, num_segments=G)
    return jnp.where(jnp.isfinite(m), m, 0.0)


def _gln(x, batch, w, b):
    cnt = jax.ops.segment_sum(jnp.ones((x.shape[0],), jnp.float32), batch, num_segments=G) * x.shape[1]
    cnt = jnp.maximum(cnt, 1.0)
    mean = jax.ops.segment_sum(jnp.sum(x, axis=1), batch, num_segments=G) / cnt
    xc = x - mean[batch][:, None]
    var = jax.ops.segment_sum(jnp.sum(xc * xc, axis=1), batch, num_segments=G) / cnt
    out = xc * jax.lax.rsqrt(var + 1e-5)[batch][:, None]
    return out * w + b


def _gat(x, src, dst, W, a_s, a_d, b, concat, n):
    h = _mm(x, W).reshape(n, H, D)
    asrc = jnp.sum(h * a_s[None], axis=-1)
    adst = jnp.sum(h * a_d[None], axis=-1)
    e = jax.nn.leaky_relu(asrc[src] + adst[dst], 0.2)
    emax = jax.ops.segment_max(e, dst, num_segments=n)
    emax = jnp.where(jnp.isfinite(emax), emax, 0.0)
    ex = jnp.exp(e - emax[dst])
    den = jax.ops.segment_sum(ex, dst, num_segments=n)
    alpha = ex / (den[dst] + 1e-16)
    out = jax.ops.segment_sum(h[src] * alpha[:, :, None], dst, num_segments=n)
    if concat:
        out = out.reshape(n, HD)
    else:
        out = out.mean(axis=1)
    return out + b


def kernel(x, edge_index, batch, params):
    p = params
    src = edge_index[0]
    dst = edge_index[1]
    n = x.shape[0]
    loop = jnp.arange(n, dtype=src.dtype)
    s2 = jnp.concatenate([src, loop])
    d2 = jnp.concatenate([dst, loop])
    h = _mm(x, p['Wg'])
    deg = jnp.zeros((n,), jnp.float32).at[d2].add(1.0)
    dinv = 1.0 / jnp.sqrt(jnp.maximum(deg, 1.0))
    coef = dinv[s2] * dinv[d2]
    y = jnp.zeros((n, D), jnp.float32).at[d2].add(h[s2] * coef[:, None]) + p['bg']
    y = jax.nn.relu(y)
    z2 = _gmax(y, batch)
    hg = _gat(x, s2, d2, p['W1'], p['as1'], p['ad1'], p['b1'], True, n)
    hg = jax.nn.relu(_gln(hg, batch, p['ln1_w'], p['ln1_b']))
    hg = _gat(hg, s2, d2, p['W2'], p['as2'], p['ad2'], p['b2'], True, n)
    hg = jax.nn.relu(_gln(hg, batch, p['ln2_w'], p['ln2_b']))
    hg = _gat(hg, s2, d2, p['W3'], p['as3'], p['ad3'], p['b3'], False, n)
    hg = jax.nn.relu(_gln(hg, batch, p['ln3_w'], p['ln3_b']))
    z1 = _gmax(hg, batch)
    z = jnp.concatenate([z1, z2], axis=1)
    t = jax.nn.relu(z @ p['l0W'] + p['l0b'])
    t = jax.nn.relu(t @ p['l1W'] + p['l1b'])
    t = jax.nn.relu(t @ p['l2W'] + p['l2b'])
    t = t @ p['lW'] + p['lb']
    return t, z


# speedup vs baseline: 21.9767x; 21.9767x over previous
import functools

import jax
import jax.numpy as jnp
from jax import lax
from jax.experimental import pallas as pl
from jax.experimental.pallas import tpu as pltpu
from jax.experimental.pallas import tpu_sc as plsc

N = 10000
F_IN = 128
H = 8
D = 32
HD = 256
G = 64
OUT = 2

NW = 32
RPW = 320
NPAD = NW * RPW
CHUNK = 3200
QCAP = 12864
B = 32
MASK14 = (1 << 14) - 1

f32 = jnp.float32
i32 = jnp.int32


def _iota16():
    return lax.broadcasted_iota(i32, (16,), 0)


def _pcount(m):
    return jnp.max(plsc.all_reduce_population_count(m))


def _full16i(x):
    return jnp.full((16,), x, i32)


def _scan_edges(src_hbm, dst_hbm, sbuf, dbuf, queue, lo, hi, nr, e_total,
                with_src, with_self):
    iota = _iota16()
    nchunk = e_total // CHUNK

    def cbody(c, qn):
        pltpu.sync_copy(dst_hbm.at[pl.ds(c * CHUNK, CHUNK)], dbuf)
        if with_src:
            pltpu.sync_copy(src_hbm.at[pl.ds(c * CHUNK, CHUNK)], sbuf)

        def jbody(j, qn):
            dv = dbuf[pl.ds(j * 16, 16)]
            m = (dv >= lo) & (dv < hi)
            dl = dv - lo
            if with_src:
                sv = sbuf[pl.ds(j * 16, 16)]
                packed = (dl << 14) | sv
            else:
                packed = dl
            plsc.store_compressed(queue.at[pl.ds(qn, 16)], packed, mask=m)
            return qn + _pcount(m)

        return lax.fori_loop(0, CHUNK // 16, jbody, qn)

    qn = lax.fori_loop(0, nchunk, cbody, 0)

    if with_self:
        def abody(k, qn):
            rv = k * 16 + iota
            m = rv < nr
            if with_src:
                packed = (rv << 14) | (lo + rv)
            else:
                packed = rv
            plsc.store_compressed(queue.at[pl.ds(qn, 16)], packed, mask=m)
            return qn + _pcount(m)

        qn = lax.fori_loop(0, RPW // 16, abody, qn)

    zv = jnp.zeros((16,), i32)
    allm = zv == zv
    plsc.store_compressed(queue.at[pl.ds(qn, 16)], zv, mask=allm)
    plsc.store_compressed(queue.at[pl.ds(qn + 16, 16)], zv, mask=allm)
    return qn


def _wid_lo():
    wid = lax.axis_index("s") * 2 + lax.axis_index("c")
    lo = wid * RPW
    nr = jnp.minimum(RPW, N - lo)
    return lo, lo + RPW, nr


def _sc_deg(e_total):
    mesh = plsc.VectorSubcoreMesh(core_axis_name="c", subcore_axis_name="s")

    @functools.partial(
        pl.kernel, mesh=mesh,
        compiler_params=pltpu.CompilerParams(needs_layout_passes=False,
                                             use_tc_tiling_on_sc=False),
        out_type=jax.ShapeDtypeStruct((NPAD,), f32),
        scratch_types=[
            pltpu.VMEM((CHUNK,), i32),
            pltpu.VMEM((QCAP + 64,), i32),
            pltpu.VMEM((RPW,), f32),
        ],
    )
    def k(dst_hbm, deg_out, dbuf, queue, dloc):
        lo, hi, nr = _wid_lo()
        iota = _iota16()

        def zbody(r, _):
            dloc[pl.ds(r * 16, 16)] = jnp.zeros((16,), f32)
            return 0
        lax.fori_loop(0, RPW // 16, zbody, 0)

        qn = _scan_edges(None, dst_hbm, None, dbuf, queue, lo, hi, nr,
                         e_total, with_src=False, with_self=False)

        lane0 = iota == 0
        ones16 = jnp.ones((16,), f32)

        def cbody(e, _):
            r = queue[pl.ds(e, 16)][0]
            plsc.addupdate_scatter(dloc, [_full16i(r)], ones16, mask=lane0)
            return 0
        lax.fori_loop(0, qn, cbody, 0)

        def fbody(kk, _):
            v = dloc[pl.ds(kk * 16, 16)]
            dloc[pl.ds(kk * 16, 16)] = v + 1.0
            return 0
        lax.fori_loop(0, RPW // 16, fbody, 0)

        pltpu.sync_copy(dloc, deg_out.at[pl.ds(lo, RPW)])

    return k


def _sc_gcn(e_total):
    mesh = plsc.VectorSubcoreMesh(core_axis_name="c", subcore_axis_name="s")
    iotac = [None, None]

    @functools.partial(
        pl.kernel, mesh=mesh,
        compiler_params=pltpu.CompilerParams(needs_layout_passes=False,
                                             use_tc_tiling_on_sc=False),
        out_type=jax.ShapeDtypeStruct((NPAD, D), f32),
        scratch_types=[
            pltpu.VMEM((CHUNK,), i32),
            pltpu.VMEM((CHUNK,), i32),
            pltpu.VMEM((QCAP + 64,), i32),
            pltpu.VMEM((RPW,), f32),
            pltpu.VMEM((RPW, D), f32),
            pltpu.VMEM((B,), i32),
            pltpu.VMEM((B,), i32),
            pltpu.VMEM((B, D), f32),
            pltpu.VMEM((B, D), f32),
            pltpu.SemaphoreType.DMA,
            pltpu.SemaphoreType.DMA,
        ],
    )
    def k(src_hbm, dst_hbm, hs_hbm, dinv_hbm, y_out,
          sbuf, dbuf, queue, dloc, acc, idx0, idx1, g0, g1, s0, s1):
        lo, hi, nr = _wid_lo()
        iota = _iota16()
        cols = [iota, iota + 16]
        fz = jnp.zeros((16,), f32)

        def zbody(r, _):
            for kk in range(2):
                plsc.store_scatter(acc, [_full16i(r), cols[kk]], fz)
            return 0
        lax.fori_loop(0, RPW, zbody, 0)
        pltpu.sync_copy(dinv_hbm.at[pl.ds(lo, RPW)], dloc)

        qn = _scan_edges(src_hbm, dst_hbm, sbuf, dbuf, queue, lo, hi, nr,
                         e_total, with_src=True, with_self=True)
        nb = (qn + B - 1) // B

        def issue(b, idxr, gb, sem):
            base = b * B
            idxr[pl.ds(0, 16)] = queue[pl.ds(base, 16)] & MASK14
            idxr[pl.ds(16, 16)] = queue[pl.ds(base + 16, 16)] & MASK14
            pltpu.async_copy(hs_hbm.at[idxr], gb, sem)

        def wait(idxr, gb, sem):
            pltpu.make_async_copy(hs_hbm.at[idxr], gb, sem).wait()

        def process(b, gb):
            cnt = jnp.minimum(B, qn - b * B)

            def ebody(i, _):
                q = queue[pl.ds(b * B + i, 16)][0]
                dl = q >> 14
                fi = _full16i(i)
                fd = _full16i(dl)
                for kk in range(2):
                    hv = plsc.load_gather(gb, [fi, cols[kk]])
                    plsc.addupdate_scatter(acc, [fd, cols[kk]], hv)
                return 0
            lax.fori_loop(0, cnt, ebody, 0)

        issue(0, idx0, g0, s0)
        nbp = (nb + 1) // 2

        def pbody(bp, _):
            b0 = bp * 2
            b1 = b0 + 1

            @pl.when(b1 < nb)
            def _():
                issue(b1, idx1, g1, s1)
            wait(idx0, g0, s0)
            process(b0, g0)

            @pl.when(b1 < nb)
            def _():
                @pl.when(b0 + 2 < nb)
                def _():
                    issue(b0 + 2, idx0, g0, s0)
                wait(idx1, g1, s1)
                process(b1, g1)
            return 0
        lax.fori_loop(0, nbp, pbody, 0)

        def fbody(kc, _):
            dvv = dloc[pl.ds(kc * 16, 16)]
            for j in range(16):
                r = kc * 16 + j
                dv = jnp.full((16,), dvv[j], f32)
                fr = _full16i(r)
                for kk in range(2):
                    a = plsc.load_gather(acc, [fr, cols[kk]]) * dv
                    plsc.store_scatter(acc, [fr, cols[kk]], a)
            return 0
        lax.fori_loop(0, RPW // 16, fbody, 0)

        pltpu.sync_copy(acc, y_out.at[pl.ds(lo, RPW)])

    return k


def _sc_gat(e_total):
    mesh = plsc.VectorSubcoreMesh(core_axis_name="c", subcore_axis_name="s")

    @functools.partial(
        pl.kernel, mesh=mesh,
        compiler_params=pltpu.CompilerParams(needs_layout_passes=False,
                                             use_tc_tiling_on_sc=False),
        out_type=jax.ShapeDtypeStruct((NPAD, HD), f32),
        scratch_types=[
            pltpu.VMEM((CHUNK,), i32),
            pltpu.VMEM((CHUNK,), i32),
            pltpu.VMEM((QCAP + 64,), i32),
            pltpu.VMEM((RPW, 16), f32),
            pltpu.VMEM((RPW, 16), f32),
            pltpu.VMEM((RPW, HD), f32),
            pltpu.VMEM((B,), i32),
            pltpu.VMEM((B,), i32),
            pltpu.VMEM((B, 16), f32),
            pltpu.VMEM((B, 16), f32),
            pltpu.VMEM((B, HD), f32),
            pltpu.VMEM((B, HD), f32),
            pltpu.SemaphoreType.DMA,
            pltpu.SemaphoreType.DMA,
            pltpu.SemaphoreType.DMA,
            pltpu.SemaphoreType.DMA,
        ],
    )
    def k(src_hbm, dst_hbm, asd_hbm, h_hbm, out_hbm,
          sbuf, dbuf, queue, asd_loc, den, acc,
          idx0, idx1, a_g0, a_g1, h_g0, h_g1,
          sa0, sa1, sh0, sh1):
        lo, hi, nr = _wid_lo()
        iota = _iota16()
        cols = [iota + 16 * kk for kk in range(16)]
        shift8 = (iota & 7) + 8
        fz = jnp.zeros((16,), f32)

        def zbody(r, _):
            fr = _full16i(r)
            for kk in range(16):
                plsc.store_scatter(acc, [fr, cols[kk]], fz)
            plsc.store_scatter(den, [fr, iota], fz)
            return 0
        lax.fori_loop(0, RPW, zbody, 0)

        pltpu.sync_copy(asd_hbm.at[pl.ds(lo, RPW)], asd_loc)

        qn = _scan_edges(src_hbm, dst_hbm, sbuf, dbuf, queue, lo, hi, nr,
                         e_total, with_src=True, with_self=True)
        nb = (qn + B - 1) // B

        def issue(b, idxr, agb, hgb, sema, semh):
            base = b * B
            idxr[pl.ds(0, 16)] = queue[pl.ds(base, 16)] & MASK14
            idxr[pl.ds(16, 16)] = queue[pl.ds(base + 16, 16)] & MASK14
            pltpu.async_copy(asd_hbm.at[idxr], agb, sema)
            pltpu.async_copy(h_hbm.at[idxr], hgb, semh)

        def wait(idxr, agb, hgb, sema, semh):
            pltpu.make_async_copy(asd_hbm.at[idxr], agb, sema).wait()
            pltpu.make_async_copy(h_hbm.at[idxr], hgb, semh).wait()

        def process(b, agb, hgb):
            cnt = jnp.minimum(B, qn - b * B)

            def ebody(i, _):
                q = queue[pl.ds(b * B + i, 16)][0]
                dl = q >> 14
                fi = _full16i(i)
                fd = _full16i(dl)
                trow = plsc.load_gather(agb, [fi, iota])
                adsh = plsc.load_gather(asd_loc, [fd, shift8])
                s = trow + adsh
                t = jnp.exp(jnp.where(s >= 0, s, 0.2 * s))
                plsc.addupdate_scatter(den, [fd, iota], t)
                tspl = [jnp.full((16,), t[hh], f32) for hh in range(8)]
                for kk in range(16):
                    hv = plsc.load_gather(hgb, [fi, cols[kk]])
                    plsc.addupdate_scatter(acc, [fd, cols[kk]],
                                           hv * tspl[kk // 2])
                return 0
            lax.fori_loop(0, cnt, ebody, 0)

        issue(0, idx0, a_g0, h_g0, sa0, sh0)
        nbp = (nb + 1) // 2

        def pbody(bp, _):
            b0 = bp * 2
            b1 = b0 + 1

            @pl.when(b1 < nb)
            def _():
                issue(b1, idx1, a_g1, h_g1, sa1, sh1)
            wait(idx0, a_g0, h_g0, sa0, sh0)
            process(b0, a_g0, h_g0)

            @pl.when(b1 < nb)
            def _():
                @pl.when(b0 + 2 < nb)
                def _():
                    issue(b0 + 2, idx0, a_g0, h_g0, sa0, sh0)
                wait(idx1, a_g1, h_g1, sa1, sh1)
                process(b1, a_g1, h_g1)
            return 0
        lax.fori_loop(0, nbp, pbody, 0)

        def fbody(r, _):
            fr = _full16i(r)
            drow = plsc.load_gather(den, [fr, iota])
            rden = 1.0 / (drow + 1e-16)
            for kk in range(16):
                dsp = jnp.full((16,), rden[kk // 2], f32)
                a = plsc.load_gather(acc, [fr, cols[kk]])
                plsc.store_scatter(acc, [fr, cols[kk]], a * dsp)
            return 0
        lax.fori_loop(0, RPW, fbody, 0)

        pltpu.sync_copy(acc, out_hbm.at[pl.ds(lo, RPW)])

    return k


BM = 400
NBLK = N // BM


def _onehot(batch_blk):
    return (batch_blk == lax.broadcasted_iota(i32, (BM, G), 1)).astype(f32)


def _stats_part(out, batch_blk):
    rs1 = jnp.sum(out, axis=1, keepdims=True)
    rs2 = jnp.sum(out * out, axis=1, keepdims=True)
    ones = jnp.ones_like(rs1)
    colsm = jnp.concatenate([rs1, rs2, ones, ones], axis=1)
    oh = _onehot(batch_blk)
    return lax.dot_general(oh, colsm, (((0,), (0,)), ((), ())),
                           preferred_element_type=f32)


def _gln_rowstats(stats, batch_blk, feat):
    cnt = jnp.maximum(stats[:, 2:3] * feat, 1.0)
    mean = stats[:, 0:1] / cnt
    var = stats[:, 1:2] / cnt - mean * mean
    rs = lax.rsqrt(var + 1e-5)
    gm = jnp.concatenate([mean, rs], axis=1)
    oh = _onehot(batch_blk)
    return jnp.dot(oh, gm, preferred_element_type=f32)


def _tc_k1_kernel(x_ref, w_ref, deg_ref, h1_ref, asd_ref, hs_ref, dinv_ref):
    o = jnp.dot(x_ref[...], w_ref[...], preferred_element_type=f32)
    h1_ref[...] = o[:, :HD]
    asd_ref[...] = o[:, HD:HD + 16]
    dinv = lax.rsqrt(jnp.maximum(deg_ref[...], 1.0))
    dinv_ref[...] = dinv
    hs_ref[...] = o[:, HD + 16:HD + 16 + D] * dinv


def _tc_post1_kernel(acc_ref, b_ref, ygcn_ref, bg_ref, batch_ref,
                     stats_ref, y_ref):
    i = pl.program_id(0)
    out = acc_ref[...] + b_ref[...]
    part = _stats_part(out, batch_ref[...])
    y_ref[...] = jnp.maximum(ygcn_ref[...] + bg_ref[...], 0.0)

    @pl.when(i == 0)
    def _():
        stats_ref[...] = part

    @pl.when(i > 0)
    def _():
        stats_ref[...] = stats_ref[...] + part


def _tc_post2_kernel(acc_ref, b_ref, batch_ref, stats_ref):
    i = pl.program_id(0)
    out = acc_ref[...] + b_ref[...]
    part = _stats_part(out, batch_ref[...])

    @pl.when(i == 0)
    def _():
        stats_ref[...] = part

    @pl.when(i > 0)
    def _():
        stats_ref[...] = stats_ref[...] + part


def _headmean(acc):
    s = acc[:, 0:D]
    for hh in range(1, H):
        s = s + acc[:, hh * D:(hh + 1) * D]
    return s * (1.0 / H)


def _tc_post3_kernel(acc_ref, b_ref, batch_ref, stats_ref):
    i = pl.program_id(0)
    out = _headmean(acc_ref[...]) + b_ref[...]
    part = _stats_part(out, batch_ref[...])

    @pl.when(i == 0)
    def _():
        stats_ref[...] = part

    @pl.when(i > 0)
    def _():
        stats_ref[...] = stats_ref[...] + part


def _tc_apply_kernel(acc_ref, b_ref, stats_ref, lnw_ref, lnb_ref, batch_ref,
                     w_ref, h_ref, asd_ref):
    out = acc_ref[...] + b_ref[...]
    mv = _gln_rowstats(stats_ref[...], batch_ref[...], HD)
    y = (out - mv[:, 0:1]) * mv[:, 1:2] * lnw_ref[...] + lnb_ref[...]
    y = jnp.maximum(y, 0.0)
    o = jnp.dot(y, w_ref[...], preferred_element_type=f32)
    h_ref[...] = o[:, :HD]
    asd_ref[...] = o[:, HD:HD + 16]


def _tc_pool_kernel(acc_ref, b_ref, stats_ref, lnw_ref, lnb_ref, batch_ref,
                    ygcn_ref, z_ref):
    i = pl.program_id(0)
    out = _headmean(acc_ref[...]) + b_ref[...]
    mv = _gln_rowstats(stats_ref[...], batch_ref[...], D)
    y3 = (out - mv[:, 0:1]) * mv[:, 1:2] * lnw_ref[...] + lnb_ref[...]
    y3 = jnp.maximum(y3, 0.0)
    yy = jnp.concatenate([y3, ygcn_ref[...]], axis=1)
    bb = batch_ref[...]
    rows = []
    for g in range(G):
        mg = bb == g
        rows.append(jnp.max(jnp.where(mg, yy, 0.0), axis=0, keepdims=True))
    zb = jnp.concatenate(rows, axis=0)

    @pl.when(i == 0)
    def _():
        z_ref[...] = zb

    @pl.when(i > 0)
    def _():
        z_ref[...] = jnp.maximum(z_ref[...], zb)


def _tc_head_kernel(z_ref, w0, b0, w1, b1, w2, b2, w3, b3, t_ref):
    t = jnp.maximum(jnp.dot(z_ref[...], w0[...],
                            preferred_element_type=f32) + b0[...], 0.0)
    t = jnp.maximum(jnp.dot(t, w1[...],
                            preferred_element_type=f32) + b1[...], 0.0)
    t = jnp.maximum(jnp.dot(t, w2[...],
                            preferred_element_type=f32) + b2[...], 0.0)
    t_ref[...] = jnp.dot(t, w3[...], preferred_element_type=f32) + b3[...]


def _rowspec(w):
    return pl.BlockSpec((BM, w), lambda i: (i, 0))


def _fullspec(a, b):
    return pl.BlockSpec((a, b), lambda i: (0, 0))


def kernel(x, edge_index, batch, params):
    p = params
    src = edge_index[0]
    dst = edge_index[1]
    e_total = src.shape[0]
    batch2d = batch.reshape(N, 1)

    def asd_w(W, a_s, a_d):
        Wr = W.reshape(W.shape[0], H, D)
        ws = jnp.einsum('fhd,hd->fh', Wr, a_s)
        wd = jnp.einsum('fhd,hd->fh', Wr, a_d)
        return jnp.concatenate([ws, wd], axis=1)

    wcat1 = jnp.concatenate([p['W1'], asd_w(p['W1'], p['as1'], p['ad1']),
                             p['Wg']], axis=1)
    wcat2 = jnp.concatenate([p['W2'], asd_w(p['W2'], p['as2'], p['ad2'])],
                            axis=1)
    wcat3 = jnp.concatenate([p['W3'], asd_w(p['W3'], p['as3'], p['ad3'])],
                            axis=1)

    deg = _sc_deg(e_total)(dst)

    h1, asd1, hs, dinv = pl.pallas_call(
        _tc_k1_kernel,
        grid=(NBLK,),
        in_specs=[_rowspec(F_IN), _fullspec(F_IN, 304), _rowspec(1)],
        out_specs=[_rowspec(HD), _rowspec(16), _rowspec(D), _rowspec(1)],
        out_shape=[jax.ShapeDtypeStruct((N, HD), f32),
                   jax.ShapeDtypeStruct((N, 16), f32),
                   jax.ShapeDtypeStruct((N, D), f32),
                   jax.ShapeDtypeStruct((N, 1), f32)],
    )(x, wcat1, deg[:N].reshape(N, 1))

    pad16 = lambda a: jnp.pad(a, ((0, NPAD - N), (0, 0)))
    dinv_p = jnp.pad(dinv[:, 0], (0, NPAD - N))

    ygcn = _sc_gcn(e_total)(src, dst, hs, dinv_p)[:N]
    acc1 = _sc_gat(e_total)(src, dst, pad16(asd1), h1)[:N]

    b1r = p['b1'].reshape(1, HD)
    bgr = p['bg'].reshape(1, D)
    stats1, y = pl.pallas_call(
        _tc_post1_kernel,
        grid=(NBLK,),
        in_specs=[_rowspec(HD), _fullspec(1, HD), _rowspec(D),
                  _fullspec(1, D), _rowspec(1)],
        out_specs=[_fullspec(G, 4), _rowspec(D)],
        out_shape=[jax.ShapeDtypeStruct((G, 4), f32),
                   jax.ShapeDtypeStruct((N, D), f32)],
    )(acc1, b1r, ygcn, bgr, batch2d)

    def apply_mm(acc, br, stats, lnw, lnb, wcat):
        return pl.pallas_call(
            _tc_apply_kernel,
            grid=(NBLK,),
            in_specs=[_rowspec(HD), _fullspec(1, HD), _fullspec(G, 4),
                      _fullspec(1, HD), _fullspec(1, HD), _rowspec(1),
                      _fullspec(HD, 272)],
            out_specs=[_rowspec(HD), _rowspec(16)],
            out_shape=[jax.ShapeDtypeStruct((N, HD), f32),
                       jax.ShapeDtypeStruct((N, 16), f32)],
        )(acc, br, stats, lnw.reshape(1, HD), lnb.reshape(1, HD), batch2d,
          wcat)

    def post_stats(accl, br):
        return pl.pallas_call(
            _tc_post2_kernel,
            grid=(NBLK,),
            in_specs=[_rowspec(HD), _fullspec(1, HD), _rowspec(1)],
            out_specs=_fullspec(G, 4),
            out_shape=jax.ShapeDtypeStruct((G, 4), f32),
        )(accl, br, batch2d)

    h2, asd2 = apply_mm(acc1, b1r, stats1, p['ln1_w'], p['ln1_b'], wcat2)
    acc2 = _sc_gat(e_total)(src, dst, pad16(asd2), h2)[:N]
    b2r = p['b2'].reshape(1, HD)
    stats2 = post_stats(acc2, b2r)

    h3, asd3 = apply_mm(acc2, b2r, stats2, p['ln2_w'], p['ln2_b'], wcat3)
    acc3 = _sc_gat(e_total)(src, dst, pad16(asd3), h3)[:N]
    b3r = p['b3'].reshape(1, D)
    stats3 = pl.pallas_call(
        _tc_post3_kernel,
        grid=(NBLK,),
        in_specs=[_rowspec(HD), _fullspec(1, D), _rowspec(1)],
        out_specs=_fullspec(G, 4),
        out_shape=jax.ShapeDtypeStruct((G, 4), f32),
    )(acc3, b3r, batch2d)

    z = pl.pallas_call(
        _tc_pool_kernel,
        grid=(NBLK,),
        in_specs=[_rowspec(HD), _fullspec(1, D), _fullspec(G, 4),
                  _fullspec(1, D), _fullspec(1, D), _rowspec(1),
                  _rowspec(D)],
        out_specs=_fullspec(G, 2 * D),
        out_shape=jax.ShapeDtypeStruct((G, 2 * D), f32),
    )(acc3, b3r, stats3, p['ln3_w'].reshape(1, D), p['ln3_b'].reshape(1, D),
      batch2d, y)

    t = pl.pallas_call(
        _tc_head_kernel,
        grid=(1,),
        in_specs=[_fullspec(G, 2 * D), _fullspec(2 * D, D), _fullspec(1, D),
                  _fullspec(D, D), _fullspec(1, D),
                  _fullspec(D, D), _fullspec(1, D),
                  _fullspec(D, OUT), _fullspec(1, OUT)],
        out_specs=_fullspec(G, OUT),
        out_shape=jax.ShapeDtypeStruct((G, OUT), f32),
    )(z, p['l0W'], p['l0b'].reshape(1, D), p['l1W'], p['l1b'].reshape(1, D),
      p['l2W'], p['l2b'].reshape(1, D), p['lW'], p['lb'].reshape(1, OUT))

    return t, z


# speedup vs baseline: 27.4316x; 1.2482x over previous
import functools

import jax
import jax.numpy as jnp
from jax import lax
from jax.experimental import pallas as pl
from jax.experimental.pallas import tpu as pltpu
from jax.experimental.pallas import tpu_sc as plsc

N = 10000
F_IN = 128
H = 8
D = 32
HD = 256
G = 64
OUT = 2
HW = HD + 16

NW = 32
RPW = 320
NPAD = NW * RPW
CHUNK = 3200
QCAP = 11264
BG = 48
BC = 64
MASK14 = (1 << 14) - 1

f32 = jnp.float32
i32 = jnp.int32


def _iota16():
    return lax.broadcasted_iota(i32, (16,), 0)


def _pcount(m):
    return plsc.all_reduce_population_count(m)[0]


def _full16i(x):
    return jnp.full((16,), x, i32)


def _wid_lo():
    wid = lax.axis_index("s") * 2 + lax.axis_index("c")
    lo = wid * RPW
    nr = jnp.minimum(RPW, N - lo)
    return wid, lo, lo + RPW, nr


def _load_queue(queues_hbm, counts_hbm, queue, cbuf, wid):
    pltpu.sync_copy(queues_hbm.at[pl.ds(wid * QCAP, QCAP)], queue)
    pltpu.sync_copy(counts_hbm.at[pl.ds(wid * 16, 16)], cbuf)
    return cbuf[...][0]


def _sc_part(e_total):
    mesh = plsc.VectorSubcoreMesh(core_axis_name="c", subcore_axis_name="s")

    @functools.partial(
        pl.kernel, mesh=mesh,
        compiler_params=pltpu.CompilerParams(needs_layout_passes=False,
                                             use_tc_tiling_on_sc=False),
        out_type=(jax.ShapeDtypeStruct((NW * QCAP,), i32),
                  jax.ShapeDtypeStruct((NW * 16,), i32),
                  jax.ShapeDtypeStruct((NPAD,), f32)),
        scratch_types=[
            pltpu.VMEM((CHUNK,), i32),
            pltpu.VMEM((CHUNK,), i32),
            pltpu.VMEM((QCAP,), i32),
            pltpu.VMEM((RPW,), f32),
            pltpu.VMEM((16,), i32),
        ],
    )
    def k(src_hbm, dst_hbm, q_out, cnt_out, deg_out,
          sbuf, dbuf, queue, dloc, cbuf):
        wid, lo, hi, nr = _wid_lo()
        iota = _iota16()

        def zbody(r, _):
            dloc[pl.ds(r * 16, 16)] = jnp.zeros((16,), f32)
            return 0
        lax.fori_loop(0, RPW // 16, zbody, 0)

        nchunk = e_total // CHUNK

        def cbody(c, qn):
            pltpu.sync_copy(dst_hbm.at[pl.ds(c * CHUNK, CHUNK)], dbuf)
            pltpu.sync_copy(src_hbm.at[pl.ds(c * CHUNK, CHUNK)], sbuf)

            def jbody(j, qn):
                dv = dbuf[pl.ds(j * 16, 16)]
                m = (dv >= lo) & (dv < hi)
                packed = ((dv - lo) << 14) | sbuf[pl.ds(j * 16, 16)]
                plsc.store_compressed(queue.at[pl.ds(qn, 16)], packed, mask=m)
                return qn + _pcount(m)

            return lax.fori_loop(0, CHUNK // 16, jbody, qn)

        qn = lax.fori_loop(0, nchunk, cbody, 0)

        def abody(kk, qn):
            rv = kk * 16 + iota
            m = rv < nr
            packed = (rv << 14) | (lo + rv)
            plsc.store_compressed(queue.at[pl.ds(qn, 16)], packed, mask=m)
            return qn + _pcount(m)
        qn = lax.fori_loop(0, RPW // 16, abody, qn)

        zv = jnp.zeros((16,), i32)
        allm = zv == zv
        plsc.store_compressed(queue.at[pl.ds(qn, 16)], zv, mask=allm)
        plsc.store_compressed(queue.at[pl.ds(qn + 16, 16)], zv, mask=allm)

        lane0 = iota == 0
        ones16 = jnp.ones((16,), f32)

        def dbody(e, _):
            q = queue[pl.ds(e, 16)][0]
            plsc.addupdate_scatter(dloc, [_full16i(q >> 14)], ones16,
                                   mask=lane0)
            return 0
        lax.fori_loop(0, qn, dbody, 0)

        pltpu.sync_copy(dloc, deg_out.at[pl.ds(lo, RPW)])
        pltpu.sync_copy(queue, q_out.at[pl.ds(wid * QCAP, QCAP)])
        cbuf[...] = _full16i(qn)
        pltpu.sync_copy(cbuf, cnt_out.at[pl.ds(wid * 16, 16)])

    return k


def _sc_gcn():
    mesh = plsc.VectorSubcoreMesh(core_axis_name="c", subcore_axis_name="s")

    @functools.partial(
        pl.kernel, mesh=mesh,
        compiler_params=pltpu.CompilerParams(needs_layout_passes=False,
                                             use_tc_tiling_on_sc=False),
        out_type=jax.ShapeDtypeStruct((NPAD, D), f32),
        scratch_types=[
            pltpu.VMEM((QCAP,), i32),
            pltpu.VMEM((16,), i32),
            pltpu.VMEM((RPW,), f32),
            pltpu.VMEM((RPW, D), f32),
            pltpu.VMEM((BC,), i32),
            pltpu.VMEM((BC,), i32),
            pltpu.VMEM((BC, D), f32),
            pltpu.VMEM((BC, D), f32),
            pltpu.SemaphoreType.DMA,
            pltpu.SemaphoreType.DMA,
        ],
    )
    def k(queues_hbm, counts_hbm, hs_hbm, dinv_hbm, y_out,
          queue, cbuf, dloc, acc, idx0, idx1, g0, g1, s0, s1):
        wid, lo, hi, nr = _wid_lo()
        iota = _iota16()
        cols = [iota, iota + 16]
        fz = jnp.zeros((16,), f32)

        def zbody(r, _):
            for kk in range(2):
                plsc.store_scatter(acc, [_full16i(r), cols[kk]], fz)
            return 0
        lax.fori_loop(0, RPW, zbody, 0)
        pltpu.sync_copy(dinv_hbm.at[pl.ds(lo, RPW)], dloc)

        qn = _load_queue(queues_hbm, counts_hbm, queue, cbuf, wid)
        nb = (qn + BC - 1) // BC

        def issue(b, idxr, gb, sem):
            base = b * BC
            for v in range(BC // 16):
                idxr[pl.ds(v * 16, 16)] = (
                    queue[pl.ds(base + v * 16, 16)] & MASK14)
            pltpu.async_copy(hs_hbm.at[idxr], gb, sem)

        def wait(idxr, gb, sem):
            pltpu.make_async_copy(hs_hbm.at[idxr], gb, sem).wait()

        def process(b, gb):
            cnt = jnp.minimum(BC, qn - b * BC)

            def ebody(i, _):
                q = queue[pl.ds(b * BC + i, 16)][0]
                fi = _full16i(i)
                fd = _full16i(q >> 14)
                for kk in range(2):
                    hv = plsc.load_gather(gb, [fi, cols[kk]])
                    plsc.addupdate_scatter(acc, [fd, cols[kk]], hv)
                return 0
            lax.fori_loop(0, cnt, ebody, 0)

        issue(0, idx0, g0, s0)
        nbp = (nb + 1) // 2

        def pbody(bp, _):
            b0 = bp * 2
            b1 = b0 + 1

            @pl.when(b1 < nb)
            def _():
                issue(b1, idx1, g1, s1)
            wait(idx0, g0, s0)
            process(b0, g0)

            @pl.when(b1 < nb)
            def _():
                @pl.when(b0 + 2 < nb)
                def _():
                    issue(b0 + 2, idx0, g0, s0)
                wait(idx1, g1, s1)
                process(b1, g1)
            return 0
        lax.fori_loop(0, nbp, pbody, 0)

        def fbody(kc, _):
            dvv = dloc[pl.ds(kc * 16, 16)]
            for j in range(16):
                r = kc * 16 + j
                dv = jnp.full((16,), dvv[j], f32)
                fr = _full16i(r)
                for kk in range(2):
                    a = plsc.load_gather(acc, [fr, cols[kk]]) * dv
                    plsc.store_scatter(acc, [fr, cols[kk]], a)
            return 0
        lax.fori_loop(0, RPW // 16, fbody, 0)

        pltpu.sync_copy(acc, y_out.at[pl.ds(lo, RPW)])

    return k


def _sc_gat():
    mesh = plsc.VectorSubcoreMesh(core_axis_name="c", subcore_axis_name="s")

    @functools.partial(
        pl.kernel, mesh=mesh,
        compiler_params=pltpu.CompilerParams(needs_layout_passes=False,
                                             use_tc_tiling_on_sc=False),
        out_type=jax.ShapeDtypeStruct((NPAD, HD), f32),
        scratch_types=[
            pltpu.VMEM((QCAP,), i32),
            pltpu.VMEM((16,), i32),
            pltpu.VMEM((RPW, 16), f32),
            pltpu.VMEM((RPW, 16), f32),
            pltpu.VMEM((RPW, HD), f32),
            pltpu.VMEM((BG,), i32),
            pltpu.VMEM((BG,), i32),
            pltpu.VMEM((BG, HW), f32),
            pltpu.VMEM((BG, HW), f32),
            pltpu.SemaphoreType.DMA,
            pltpu.SemaphoreType.DMA,
        ],
    )
    def k(queues_hbm, counts_hbm, asd_hbm, h_hbm, out_hbm,
          queue, cbuf, asd_loc, den, acc, idx0, idx1, h_g0, h_g1, sh0, sh1):
        wid, lo, hi, nr = _wid_lo()
        iota = _iota16()
        cols = [iota + 16 * kk for kk in range(16)]
        acol = iota + HD
        shift8 = (iota & 7) + 8
        fz = jnp.zeros((16,), f32)

        def zbody(r, _):
            fr = _full16i(r)
            for kk in range(16):
                plsc.store_scatter(acc, [fr, cols[kk]], fz)
            plsc.store_scatter(den, [fr, iota], fz)
            return 0
        lax.fori_loop(0, RPW, zbody, 0)

        pltpu.sync_copy(asd_hbm.at[pl.ds(lo, RPW)], asd_loc)
        qn = _load_queue(queues_hbm, counts_hbm, queue, cbuf, wid)
        nb = (qn + BG - 1) // BG

        def issue(b, idxr, hgb, semh):
            base = b * BG
            for v in range(BG // 16):
                idxr[pl.ds(v * 16, 16)] = (
                    queue[pl.ds(base + v * 16, 16)] & MASK14)
            pltpu.async_copy(h_hbm.at[idxr], hgb, semh)

        def wait(idxr, hgb, semh):
            pltpu.make_async_copy(h_hbm.at[idxr], hgb, semh).wait()

        def process(b, hgb):
            cnt = jnp.minimum(BG, qn - b * BG)

            def ebody(i, _):
                q = queue[pl.ds(b * BG + i, 16)][0]
                dl = q >> 14
                fi = _full16i(i)
                fd = _full16i(dl)
                trow = plsc.load_gather(hgb, [fi, acol])
                adsh = plsc.load_gather(asd_loc, [fd, shift8])
                s = trow + adsh
                t = jnp.exp(jnp.where(s >= 0, s, 0.2 * s))
                plsc.addupdate_scatter(den, [fd, iota], t)
                tspl = [jnp.full((16,), t[hh], f32) for hh in range(8)]
                for kk in range(16):
                    hv = plsc.load_gather(hgb, [fi, cols[kk]])
                    plsc.addupdate_scatter(acc, [fd, cols[kk]],
                                           hv * tspl[kk // 2])
                return 0
            lax.fori_loop(0, cnt, ebody, 0)

        issue(0, idx0, h_g0, sh0)
        nbp = (nb + 1) // 2

        def pbody(bp, _):
            b0 = bp * 2
            b1 = b0 + 1

            @pl.when(b1 < nb)
            def _():
                issue(b1, idx1, h_g1, sh1)
            wait(idx0, h_g0, sh0)
            process(b0, h_g0)

            @pl.when(b1 < nb)
            def _():
                @pl.when(b0 + 2 < nb)
                def _():
                    issue(b0 + 2, idx0, h_g0, sh0)
                wait(idx1, h_g1, sh1)
                process(b1, h_g1)
            return 0
        lax.fori_loop(0, nbp, pbody, 0)

        def fbody(r, _):
            fr = _full16i(r)
            drow = plsc.load_gather(den, [fr, iota])
            rden = 1.0 / (drow + 1e-16)
            for kk in range(16):
                dsp = jnp.full((16,), rden[kk // 2], f32)
                a = plsc.load_gather(acc, [fr, cols[kk]])
                plsc.store_scatter(acc, [fr, cols[kk]], a * dsp)
            return 0
        lax.fori_loop(0, RPW, fbody, 0)

        pltpu.sync_copy(acc, out_hbm.at[pl.ds(lo, RPW)])

    return k


BM = 400
NBLK = N // BM


def _onehot(batch_blk):
    return (batch_blk == lax.broadcasted_iota(i32, (BM, G), 1)).astype(f32)


def _stats_part(out, batch_blk):
    rs1 = jnp.sum(out, axis=1, keepdims=True)
    rs2 = jnp.sum(out * out, axis=1, keepdims=True)
    ones = jnp.ones_like(rs1)
    colsm = jnp.concatenate([rs1, rs2, ones, ones], axis=1)
    oh = _onehot(batch_blk)
    return lax.dot_general(oh, colsm, (((0,), (0,)), ((), ())),
                           preferred_element_type=f32)


def _gln_rowstats(stats, batch_blk, feat):
    cnt = jnp.maximum(stats[:, 2:3] * feat, 1.0)
    mean = stats[:, 0:1] / cnt
    var = stats[:, 1:2] / cnt - mean * mean
    rs = lax.rsqrt(var + 1e-5)
    gm = jnp.concatenate([mean, rs], axis=1)
    oh = _onehot(batch_blk)
    return jnp.dot(oh, gm, preferred_element_type=f32)


def _tc_k1_kernel(x_ref, w_ref, deg_ref, h1_ref, asd_ref, hs_ref, dinv_ref):
    o = jnp.dot(x_ref[...], w_ref[...], preferred_element_type=f32)
    h1_ref[...] = o[:, :HW]
    asd_ref[...] = o[:, HD:HW]
    dinv = lax.rsqrt(jnp.maximum(deg_ref[...], 1.0))
    dinv_ref[...] = dinv
    hs_ref[...] = o[:, HW:HW + D] * dinv


def _tc_post1_kernel(acc_ref, b_ref, ygcn_ref, bg_ref, batch_ref,
                     stats_ref, y_ref):
    i = pl.program_id(0)
    out = acc_ref[...] + b_ref[...]
    part = _stats_part(out, batch_ref[...])
    y_ref[...] = jnp.maximum(ygcn_ref[...] + bg_ref[...], 0.0)

    @pl.when(i == 0)
    def _():
        stats_ref[...] = part

    @pl.when(i > 0)
    def _():
        stats_ref[...] = stats_ref[...] + part


def _tc_post2_kernel(acc_ref, b_ref, batch_ref, stats_ref):
    i = pl.program_id(0)
    out = acc_ref[...] + b_ref[...]
    part = _stats_part(out, batch_ref[...])

    @pl.when(i == 0)
    def _():
        stats_ref[...] = part

    @pl.when(i > 0)
    def _():
        stats_ref[...] = stats_ref[...] + part


def _headmean(acc):
    s = acc[:, 0:D]
    for hh in range(1, H):
        s = s + acc[:, hh * D:(hh + 1) * D]
    return s * (1.0 / H)


def _tc_post3_kernel(acc_ref, b_ref, batch_ref, stats_ref):
    i = pl.program_id(0)
    out = _headmean(acc_ref[...]) + b_ref[...]
    part = _stats_part(out, batch_ref[...])

    @pl.when(i == 0)
    def _():
        stats_ref[...] = part

    @pl.when(i > 0)
    def _():
        stats_ref[...] = stats_ref[...] + part


def _tc_apply_kernel(acc_ref, b_ref, stats_ref, lnw_ref, lnb_ref, batch_ref,
                     w_ref, h_ref, asd_ref):
    out = acc_ref[...] + b_ref[...]
    mv = _gln_rowstats(stats_ref[...], batch_ref[...], HD)
    y = (out - mv[:, 0:1]) * mv[:, 1:2] * lnw_ref[...] + lnb_ref[...]
    y = jnp.maximum(y, 0.0)
    o = jnp.dot(y, w_ref[...], preferred_element_type=f32)
    h_ref[...] = o
    asd_ref[...] = o[:, HD:HW]


def _tc_pool_kernel(acc_ref, b_ref, stats_ref, lnw_ref, lnb_ref, batch_ref,
                    ygcn_ref, z_ref):
    i = pl.program_id(0)
    out = _headmean(acc_ref[...]) + b_ref[...]
    mv = _gln_rowstats(stats_ref[...], batch_ref[...], D)
    y3 = (out - mv[:, 0:1]) * mv[:, 1:2] * lnw_ref[...] + lnb_ref[...]
    y3 = jnp.maximum(y3, 0.0)
    yy = jnp.concatenate([y3, ygcn_ref[...]], axis=1)
    bb = batch_ref[...]
    rows = []
    for g in range(G):
        mg = bb == g
        rows.append(jnp.max(jnp.where(mg, yy, 0.0), axis=0, keepdims=True))
    zb = jnp.concatenate(rows, axis=0)

    @pl.when(i == 0)
    def _():
        z_ref[...] = zb

    @pl.when(i > 0)
    def _():
        z_ref[...] = jnp.maximum(z_ref[...], zb)


def _tc_head_kernel(z_ref, w0, b0, w1, b1, w2, b2, w3, b3, t_ref):
    t = jnp.maximum(jnp.dot(z_ref[...], w0[...],
                            preferred_element_type=f32) + b0[...], 0.0)
    t = jnp.maximum(jnp.dot(t, w1[...],
                            preferred_element_type=f32) + b1[...], 0.0)
    t = jnp.maximum(jnp.dot(t, w2[...],
                            preferred_element_type=f32) + b2[...], 0.0)
    t_ref[...] = jnp.dot(t, w3[...], preferred_element_type=f32) + b3[...]


def _rowspec(w):
    return pl.BlockSpec((BM, w), lambda i: (i, 0))


def _fullspec(a, b):
    return pl.BlockSpec((a, b), lambda i: (0, 0))


def kernel(x, edge_index, batch, params):
    p = params
    src = edge_index[0]
    dst = edge_index[1]
    e_total = src.shape[0]
    batch2d = batch.reshape(N, 1)

    def asd_w(W, a_s, a_d):
        Wr = W.reshape(W.shape[0], H, D)
        ws = jnp.einsum('fhd,hd->fh', Wr, a_s)
        wd = jnp.einsum('fhd,hd->fh', Wr, a_d)
        return jnp.concatenate([ws, wd], axis=1)

    wcat1 = jnp.concatenate([p['W1'], asd_w(p['W1'], p['as1'], p['ad1']),
                             p['Wg']], axis=1)
    wcat2 = jnp.concatenate([p['W2'], asd_w(p['W2'], p['as2'], p['ad2'])],
                            axis=1)
    wcat3 = jnp.concatenate([p['W3'], asd_w(p['W3'], p['as3'], p['ad3'])],
                            axis=1)

    queues, counts, deg = _sc_part(e_total)(src, dst)

    h1, asd1, hs, dinv = pl.pallas_call(
        _tc_k1_kernel,
        grid=(NBLK,),
        in_specs=[_rowspec(F_IN), _fullspec(F_IN, 304), _rowspec(1)],
        out_specs=[_rowspec(HW), _rowspec(16), _rowspec(D), _rowspec(1)],
        out_shape=[jax.ShapeDtypeStruct((N, HW), f32),
                   jax.ShapeDtypeStruct((N, 16), f32),
                   jax.ShapeDtypeStruct((N, D), f32),
                   jax.ShapeDtypeStruct((N, 1), f32)],
    )(x, wcat1, deg[:N].reshape(N, 1))

    pad16 = lambda a: jnp.pad(a, ((0, NPAD - N), (0, 0)))
    dinv_p = jnp.pad(dinv[:, 0], (0, NPAD - N))

    ygcn = _sc_gcn()(queues, counts, hs, dinv_p)[:N]
    acc1 = _sc_gat()(queues, counts, pad16(asd1), h1)[:N]

    b1r = p['b1'].reshape(1, HD)
    bgr = p['bg'].reshape(1, D)
    stats1, y = pl.pallas_call(
        _tc_post1_kernel,
        grid=(NBLK,),
        in_specs=[_rowspec(HD), _fullspec(1, HD), _rowspec(D),
                  _fullspec(1, D), _rowspec(1)],
        out_specs=[_fullspec(G, 4), _rowspec(D)],
        out_shape=[jax.ShapeDtypeStruct((G, 4), f32),
                   jax.ShapeDtypeStruct((N, D), f32)],
    )(acc1, b1r, ygcn, bgr, batch2d)

    def apply_mm(acc, br, stats, lnw, lnb, wcat):
        return pl.pallas_call(
            _tc_apply_kernel,
            grid=(NBLK,),
            in_specs=[_rowspec(HD), _fullspec(1, HD), _fullspec(G, 4),
                      _fullspec(1, HD), _fullspec(1, HD), _rowspec(1),
                      _fullspec(HD, HW)],
            out_specs=[_rowspec(HW), _rowspec(16)],
            out_shape=[jax.ShapeDtypeStruct((N, HW), f32),
                       jax.ShapeDtypeStruct((N, 16), f32)],
        )(acc, br, stats, lnw.reshape(1, HD), lnb.reshape(1, HD), batch2d,
          wcat)

    h2, asd2 = apply_mm(acc1, b1r, stats1, p['ln1_w'], p['ln1_b'], wcat2)
    acc2 = _sc_gat()(queues, counts, pad16(asd2), h2)[:N]
    b2r = p['b2'].reshape(1, HD)
    stats2 = pl.pallas_call(
        _tc_post2_kernel,
        grid=(NBLK,),
        in_specs=[_rowspec(HD), _fullspec(1, HD), _rowspec(1)],
        out_specs=_fullspec(G, 4),
        out_shape=jax.ShapeDtypeStruct((G, 4), f32),
    )(acc2, b2r, batch2d)

    h3, asd3 = apply_mm(acc2, b2r, stats2, p['ln2_w'], p['ln2_b'], wcat3)
    acc3 = _sc_gat()(queues, counts, pad16(asd3), h3)[:N]
    b3r = p['b3'].reshape(1, D)
    stats3 = pl.pallas_call(
        _tc_post3_kernel,
        grid=(NBLK,),
        in_specs=[_rowspec(HD), _fullspec(1, D), _rowspec(1)],
        out_specs=_fullspec(G, 4),
        out_shape=jax.ShapeDtypeStruct((G, 4), f32),
    )(acc3, b3r, batch2d)

    z = pl.pallas_call(
        _tc_pool_kernel,
        grid=(NBLK,),
        in_specs=[_rowspec(HD), _fullspec(1, D), _fullspec(G, 4),
                  _fullspec(1, D), _fullspec(1, D), _rowspec(1),
                  _rowspec(D)],
        out_specs=_fullspec(G, 2 * D),
        out_shape=jax.ShapeDtypeStruct((G, 2 * D), f32),
    )(acc3, b3r, stats3, p['ln3_w'].reshape(1, D), p['ln3_b'].reshape(1, D),
      batch2d, y)

    t = pl.pallas_call(
        _tc_head_kernel,
        grid=(1,),
        in_specs=[_fullspec(G, 2 * D), _fullspec(2 * D, D), _fullspec(1, D),
                  _fullspec(D, D), _fullspec(1, D),
                  _fullspec(D, D), _fullspec(1, D),
                  _fullspec(D, OUT), _fullspec(1, OUT)],
        out_specs=_fullspec(G, OUT),
        out_shape=jax.ShapeDtypeStruct((G, OUT), f32),
    )(z, p['l0W'], p['l0b'].reshape(1, D), p['l1W'], p['l1b'].reshape(1, D),
      p['l2W'], p['l2b'].reshape(1, D), p['lW'], p['lb'].reshape(1, OUT))

    return t, z
